# Initial kernel scaffold; baseline (speedup 1.0000x reference)
import functools

import jax
import jax.numpy as jnp
from jax import lax
from jax.experimental import pallas as pl
from jax.experimental.pallas import tpu as pltpu
from jax.experimental.pallas import tpu_sc as plsc

N = 10000
NNZ = 160000
C = 256
H = 128
NCORES = 2
NTILES = 16
B = 128
NCHUNK = 80
PER_TILE = NCHUNK * B
NNZ_PAD = NTILES * PER_TILE
APAD = 10240
RPT = APAD // NTILES

_MESH = plsc.VectorSubcoreMesh(
    core_axis_name="c", subcore_axis_name="s",
    num_cores=NCORES, num_subcores=NTILES)


def _mm_body(x_ref, w_ref, b_ref, oa_ref, ob_ref):
    acc = jnp.dot(x_ref[...], w_ref[...],
                  preferred_element_type=jnp.float32) + b_ref[...]
    oa_ref[...] = acc[:, :H]
    ob_ref[...] = acc[:, H:]


def _matmul(x, w, b2):
    return pl.pallas_call(
        _mm_body,
        grid=(10,),
        in_specs=[
            pl.BlockSpec((1000, C), lambda i: (i, 0)),
            pl.BlockSpec((C, C), lambda i: (0, 0)),
            pl.BlockSpec((1, C), lambda i: (0, 0)),
        ],
        out_specs=[pl.BlockSpec((1000, H), lambda i: (i, 0))] * 2,
        out_shape=[jax.ShapeDtypeStruct((N, H), jnp.float32)] * 2,
    )(x, w, b2)


def _stream_loop(table, gidx_v, sidx_v, rows0, rows1, acc, sem0, sem1,
                 cnt_add=None):
    pltpu.async_copy(table.at[gidx_v.at[0]], rows0, sem0)
    pltpu.async_copy(table.at[gidx_v.at[1]], rows1, sem1)

    @pl.loop(0, NCHUNK, step=2)
    def _(j):
        for k, (rows, sem) in enumerate(((rows0, sem0), (rows1, sem1))):
            jj = j + k
            pltpu.make_async_copy(table.at[gidx_v.at[jj]], rows, sem).wait()
            pltpu.sync_copy(rows, acc.at[sidx_v.at[jj]], add=True)
            if cnt_add is not None:
                ones_v, cnt_acc = cnt_add
                pltpu.sync_copy(ones_v, cnt_acc.at[sidx_v.at[jj]], add=True)

            @pl.when(jj + 2 < NCHUNK)
            def _():
                pltpu.async_copy(table.at[gidx_v.at[jj + 2]], rows, sem)


def _v2e_body(ta, tb, gidx_h, sidx_h, zrow, zcnt, ones_h,
              o_a, o_b, o_cnt,
              gidx_v, sidx_v, rows0, rows1, ones_v, acc, cnt_acc, sem0, sem1):
    cid = lax.axis_index("c")
    sid = lax.axis_index("s")
    r0 = sid * RPT
    pltpu.sync_copy(zrow, acc.at[pl.ds(r0, RPT)])
    pltpu.sync_copy(gidx_h.at[sid], gidx_v)
    pltpu.sync_copy(sidx_h.at[sid], sidx_v)

    @pl.when(cid == 0)
    def _():
        pltpu.sync_copy(zcnt, cnt_acc.at[pl.ds(r0, RPT)])
        pltpu.sync_copy(ones_h, ones_v)

    plsc.subcore_barrier()

    @pl.when(cid == 0)
    def _():
        _stream_loop(ta, gidx_v, sidx_v, rows0, rows1, acc, sem0, sem1,
                     cnt_add=(ones_v, cnt_acc))

    @pl.when(cid == 1)
    def _():
        _stream_loop(tb, gidx_v, sidx_v, rows0, rows1, acc, sem0, sem1)

    plsc.subcore_barrier()

    @pl.when(cid == 0)
    def _():
        pltpu.sync_copy(acc.at[pl.ds(r0, RPT)], o_a.at[pl.ds(r0, RPT)])
        pltpu.sync_copy(cnt_acc.at[pl.ds(r0, RPT)], o_cnt.at[pl.ds(r0, RPT)])

    @pl.when(cid == 1)
    def _():
        pltpu.sync_copy(acc.at[pl.ds(r0, RPT)], o_b.at[pl.ds(r0, RPT)])


_v2e = functools.partial(
    pl.kernel,
    out_type=[
        jax.ShapeDtypeStruct((APAD, H), jnp.float32),
        jax.ShapeDtypeStruct((APAD, H), jnp.float32),
        jax.ShapeDtypeStruct((APAD, 16), jnp.float32),
    ],
    mesh=_MESH,
    scratch_types=[
        pltpu.VMEM((NCHUNK, B), jnp.int32),
        pltpu.VMEM((NCHUNK, B), jnp.int32),
        pltpu.VMEM((B, H), jnp.float32),
        pltpu.VMEM((B, H), jnp.float32),
        pltpu.VMEM((B, 16), jnp.float32),
        pltpu.VMEM_SHARED((APAD, H), jnp.float32),
        pltpu.VMEM_SHARED((APAD, 16), jnp.float32),
        pltpu.SemaphoreType.DMA,
        pltpu.SemaphoreType.DMA,
    ],
)(_v2e_body)


def _e2v_body(ta, tb, gidx_h, sidx_h, zrow,
              o_a, o_b,
              gidx_v, sidx_v, rows0, rows1, acc, sem0, sem1):
    cid = lax.axis_index("c")
    sid = lax.axis_index("s")
    r0 = sid * RPT
    pltpu.sync_copy(zrow, acc.at[pl.ds(r0, RPT)])
    pltpu.sync_copy(gidx_h.at[sid], gidx_v)
    pltpu.sync_copy(sidx_h.at[sid], sidx_v)

    plsc.subcore_barrier()

    @pl.when(cid == 0)
    def _():
        _stream_loop(ta, gidx_v, sidx_v, rows0, rows1, acc, sem0, sem1)

    @pl.when(cid == 1)
    def _():
        _stream_loop(tb, gidx_v, sidx_v, rows0, rows1, acc, sem0, sem1)

    plsc.subcore_barrier()

    @pl.when(cid == 0)
    def _():
        pltpu.sync_copy(acc.at[pl.ds(r0, RPT)], o_a.at[pl.ds(r0, RPT)])

    @pl.when(cid == 1)
    def _():
        pltpu.sync_copy(acc.at[pl.ds(r0, RPT)], o_b.at[pl.ds(r0, RPT)])


_e2v = functools.partial(
    pl.kernel,
    out_type=[
        jax.ShapeDtypeStruct((APAD, H), jnp.float32),
        jax.ShapeDtypeStruct((APAD, H), jnp.float32),
    ],
    mesh=_MESH,
    scratch_types=[
        pltpu.VMEM((NCHUNK, B), jnp.int32),
        pltpu.VMEM((NCHUNK, B), jnp.int32),
        pltpu.VMEM((B, H), jnp.float32),
        pltpu.VMEM((B, H), jnp.float32),
        pltpu.VMEM_SHARED((APAD, H), jnp.float32),
        pltpu.SemaphoreType.DMA,
        pltpu.SemaphoreType.DMA,
    ],
)(_e2v_body)


def _scale_body(sa_ref, sb_ref, cnt_ref, ya_ref, yb_ref):
    inv = 1.0 / jnp.maximum(cnt_ref[:, 0:1], 1.0)
    ya_ref[...] = sa_ref[...] * inv
    yb_ref[...] = sb_ref[...] * inv


def _scale(sa, sb, cnt):
    return pl.pallas_call(
        _scale_body,
        grid=(10,),
        in_specs=[
            pl.BlockSpec((1024, H), lambda i: (i, 0)),
            pl.BlockSpec((1024, H), lambda i: (i, 0)),
            pl.BlockSpec((1024, 16), lambda i: (i, 0)),
        ],
        out_specs=[pl.BlockSpec((1024, H), lambda i: (i, 0))] * 2,
        out_shape=[jax.ShapeDtypeStruct((APAD, H), jnp.float32)] * 2,
    )(sa, sb, cnt)


def _final_body(aa_ref, ab_ref, xa_ref, xb_ref, o_ref):
    o_ref[:, :H] = jnp.maximum(aa_ref[...] + xa_ref[...], 0.0)
    o_ref[:, H:] = jnp.maximum(ab_ref[...] + xb_ref[...], 0.0)


def _final(aa, ab, xa, xb):
    return pl.pallas_call(
        _final_body,
        grid=(10,),
        in_specs=[pl.BlockSpec((1000, H), lambda i: (i, 0))] * 4,
        out_specs=pl.BlockSpec((1000, C), lambda i: (i, 0)),
        out_shape=jax.ShapeDtypeStruct((N, C), jnp.float32),
    )(aa, ab, xa, xb)


def kernel(X, hyperedge_index, W, b):
    v = hyperedge_index[0].astype(jnp.int32)
    e = hyperedge_index[1].astype(jnp.int32)
    pad = NNZ_PAD - NNZ
    v_p = jnp.concatenate([v, jnp.zeros((pad,), jnp.int32)])
    v_p = v_p.reshape(NTILES, NCHUNK, B)
    e_p = jnp.concatenate([e, jnp.full((pad,), N, jnp.int32)])
    e_p = e_p.reshape(NTILES, NCHUNK, B)

    zrow = jnp.zeros((RPT, H), jnp.float32)
    zcnt = jnp.zeros((RPT, 16), jnp.float32)
    ones = jnp.ones((B, 16), jnp.float32)

    xt_a, xt_b = _matmul(X, W, b.reshape(1, C))
    sums_a, sums_b, cnt = _v2e(xt_a, xt_b, v_p, e_p, zrow, zcnt, ones)
    y_a, y_b = _scale(sums_a, sums_b, cnt)
    agg_a, agg_b = _e2v(y_a, y_b, e_p, v_p, zrow)
    return _final(agg_a, agg_b, xt_a, xt_b)


# speedup vs baseline: 3.0545x; 3.0545x over previous
import functools

import jax
import jax.numpy as jnp
from jax import lax
from jax.experimental import pallas as pl
from jax.experimental.pallas import tpu as pltpu
from jax.experimental.pallas import tpu_sc as plsc

N = 10000
NNZ = 160000
C = 256
G = 64
NCORES = 2
NTILES = 16
B = 128
NCHUNK = 80
PER_TILE = NCHUNK * B
NNZ_PAD = NTILES * PER_TILE
APAD = 10240
RPT = APAD // NTILES

_MESH = plsc.VectorSubcoreMesh(
    core_axis_name="c", subcore_axis_name="s",
    num_cores=NCORES, num_subcores=NTILES)


def _mm_body(x_ref, w_ref, b_ref, o0_ref, o1_ref, o2_ref, o3_ref):
    acc = jnp.dot(x_ref[...], w_ref[...],
                  preferred_element_type=jnp.float32) + b_ref[...]
    o0_ref[...] = acc[:, 0 * G:1 * G]
    o1_ref[...] = acc[:, 1 * G:2 * G]
    o2_ref[...] = acc[:, 2 * G:3 * G]
    o3_ref[...] = acc[:, 3 * G:4 * G]


def _matmul(x, w, b2):
    return pl.pallas_call(
        _mm_body,
        grid=(10,),
        in_specs=[
            pl.BlockSpec((1000, C), lambda i: (i, 0)),
            pl.BlockSpec((C, C), lambda i: (0, 0)),
            pl.BlockSpec((1, C), lambda i: (0, 0)),
        ],
        out_specs=[pl.BlockSpec((1000, G), lambda i: (i, 0))] * 4,
        out_shape=[jax.ShapeDtypeStruct((N, G), jnp.float32)] * 4,
    )(x, w, b2)


def _stream_loop(table, gidx_v, sidx_v, rows0, rows1, acc, sem0, sem1,
                 cnt_add=None):
    pltpu.async_copy(table.at[gidx_v.at[0]], rows0, sem0)
    pltpu.async_copy(table.at[gidx_v.at[1]], rows1, sem1)

    @pl.loop(0, NCHUNK, step=2)
    def _(j):
        for k, (rows, sem) in enumerate(((rows0, sem0), (rows1, sem1))):
            jj = j + k
            pltpu.make_async_copy(table.at[gidx_v.at[jj]], rows, sem).wait()
            pltpu.sync_copy(rows, acc.at[sidx_v.at[jj]], add=True)
            if cnt_add is not None:
                ones_v, cnt_acc = cnt_add
                pltpu.sync_copy(ones_v, cnt_acc.at[sidx_v.at[jj]], add=True)

            @pl.when(jj + 2 < NCHUNK)
            def _():
                pltpu.async_copy(table.at[gidx_v.at[jj + 2]], rows, sem)


def _zero_acc(zrow, acc, r0):
    pltpu.sync_copy(zrow, acc.at[pl.ds(r0, RPT)])


def _drain_acc(acc, out_hbm, r0):
    pltpu.sync_copy(acc.at[pl.ds(r0, RPT)], out_hbm.at[pl.ds(r0, RPT)])


def _v2e_body(t0, t1, t2, t3, gidx_h, sidx_h, zrow, zcnt, ones_h,
              o0, o1, o2, o3, o_cnt,
              gidx_v, sidx_v, rows0, rows1, ones_v, acc, cnt_acc, sem0, sem1):
    cid = lax.axis_index("c")
    sid = lax.axis_index("s")
    r0 = sid * RPT
    _zero_acc(zrow, acc, r0)
    pltpu.sync_copy(gidx_h.at[sid], gidx_v)
    pltpu.sync_copy(sidx_h.at[sid], sidx_v)

    @pl.when(cid == 0)
    def _():
        pltpu.sync_copy(zcnt, cnt_acc.at[pl.ds(r0, RPT)])
        pltpu.sync_copy(ones_h, ones_v)

    plsc.subcore_barrier()

    @pl.when(cid == 0)
    def _():
        _stream_loop(t0, gidx_v, sidx_v, rows0, rows1, acc, sem0, sem1,
                     cnt_add=(ones_v, cnt_acc))

    @pl.when(cid == 1)
    def _():
        _stream_loop(t2, gidx_v, sidx_v, rows0, rows1, acc, sem0, sem1)

    plsc.subcore_barrier()

    @pl.when(cid == 0)
    def _():
        _drain_acc(acc, o0, r0)
        pltpu.sync_copy(cnt_acc.at[pl.ds(r0, RPT)], o_cnt.at[pl.ds(r0, RPT)])

    @pl.when(cid == 1)
    def _():
        _drain_acc(acc, o2, r0)

    plsc.subcore_barrier()

    _zero_acc(zrow, acc, r0)
    plsc.subcore_barrier()

    @pl.when(cid == 0)
    def _():
        _stream_loop(t1, gidx_v, sidx_v, rows0, rows1, acc, sem0, sem1)

    @pl.when(cid == 1)
    def _():
        _stream_loop(t3, gidx_v, sidx_v, rows0, rows1, acc, sem0, sem1)

    plsc.subcore_barrier()

    @pl.when(cid == 0)
    def _():
        _drain_acc(acc, o1, r0)

    @pl.when(cid == 1)
    def _():
        _drain_acc(acc, o3, r0)


_v2e = functools.partial(
    pl.kernel,
    out_type=[jax.ShapeDtypeStruct((APAD, G), jnp.float32)] * 4
    + [jax.ShapeDtypeStruct((APAD, 16), jnp.float32)],
    mesh=_MESH,
    compiler_params=pltpu.CompilerParams(use_tc_tiling_on_sc=False),
    scratch_types=[
        pltpu.VMEM((NCHUNK, B), jnp.int32),
        pltpu.VMEM((NCHUNK, B), jnp.int32),
        pltpu.VMEM((B, G), jnp.float32),
        pltpu.VMEM((B, G), jnp.float32),
        pltpu.VMEM((B, 16), jnp.float32),
        pltpu.VMEM_SHARED((APAD, G), jnp.float32),
        pltpu.VMEM_SHARED((APAD, 16), jnp.float32),
        pltpu.SemaphoreType.DMA,
        pltpu.SemaphoreType.DMA,
    ],
)(_v2e_body)


def _e2v_body(t0, t1, t2, t3, gidx_h, sidx_h, zrow,
              o0, o1, o2, o3,
              gidx_v, sidx_v, rows0, rows1, acc, sem0, sem1):
    cid = lax.axis_index("c")
    sid = lax.axis_index("s")
    r0 = sid * RPT
    _zero_acc(zrow, acc, r0)
    pltpu.sync_copy(gidx_h.at[sid], gidx_v)
    pltpu.sync_copy(sidx_h.at[sid], sidx_v)

    plsc.subcore_barrier()

    @pl.when(cid == 0)
    def _():
        _stream_loop(t0, gidx_v, sidx_v, rows0, rows1, acc, sem0, sem1)

    @pl.when(cid == 1)
    def _():
        _stream_loop(t2, gidx_v, sidx_v, rows0, rows1, acc, sem0, sem1)

    plsc.subcore_barrier()

    @pl.when(cid == 0)
    def _():
        _drain_acc(acc, o0, r0)

    @pl.when(cid == 1)
    def _():
        _drain_acc(acc, o2, r0)

    plsc.subcore_barrier()

    _zero_acc(zrow, acc, r0)
    plsc.subcore_barrier()

    @pl.when(cid == 0)
    def _():
        _stream_loop(t1, gidx_v, sidx_v, rows0, rows1, acc, sem0, sem1)

    @pl.when(cid == 1)
    def _():
        _stream_loop(t3, gidx_v, sidx_v, rows0, rows1, acc, sem0, sem1)

    plsc.subcore_barrier()

    @pl.when(cid == 0)
    def _():
        _drain_acc(acc, o1, r0)

    @pl.when(cid == 1)
    def _():
        _drain_acc(acc, o3, r0)


_e2v = functools.partial(
    pl.kernel,
    out_type=[jax.ShapeDtypeStruct((APAD, G), jnp.float32)] * 4,
    mesh=_MESH,
    compiler_params=pltpu.CompilerParams(use_tc_tiling_on_sc=False),
    scratch_types=[
        pltpu.VMEM((NCHUNK, B), jnp.int32),
        pltpu.VMEM((NCHUNK, B), jnp.int32),
        pltpu.VMEM((B, G), jnp.float32),
        pltpu.VMEM((B, G), jnp.float32),
        pltpu.VMEM_SHARED((APAD, G), jnp.float32),
        pltpu.SemaphoreType.DMA,
        pltpu.SemaphoreType.DMA,
    ],
)(_e2v_body)


def _scale_body(s0, s1, s2, s3, cnt_ref, y0, y1, y2, y3):
    inv = 1.0 / jnp.maximum(cnt_ref[:, 0:1], 1.0)
    y0[...] = s0[...] * inv
    y1[...] = s1[...] * inv
    y2[...] = s2[...] * inv
    y3[...] = s3[...] * inv


def _scale(s0, s1, s2, s3, cnt):
    return pl.pallas_call(
        _scale_body,
        grid=(10,),
        in_specs=[pl.BlockSpec((1024, G), lambda i: (i, 0))] * 4
        + [pl.BlockSpec((1024, 16), lambda i: (i, 0))],
        out_specs=[pl.BlockSpec((1024, G), lambda i: (i, 0))] * 4,
        out_shape=[jax.ShapeDtypeStruct((APAD, G), jnp.float32)] * 4,
    )(s0, s1, s2, s3, cnt)


def _final_body(a0, a1, a2, a3, x0, x1, x2, x3, o_ref):
    o_ref[:, 0 * G:1 * G] = jnp.maximum(a0[...] + x0[...], 0.0)
    o_ref[:, 1 * G:2 * G] = jnp.maximum(a1[...] + x1[...], 0.0)
    o_ref[:, 2 * G:3 * G] = jnp.maximum(a2[...] + x2[...], 0.0)
    o_ref[:, 3 * G:4 * G] = jnp.maximum(a3[...] + x3[...], 0.0)


def _final(aggs, xts):
    return pl.pallas_call(
        _final_body,
        grid=(10,),
        in_specs=[pl.BlockSpec((1000, G), lambda i: (i, 0))] * 8,
        out_specs=pl.BlockSpec((1000, C), lambda i: (i, 0)),
        out_shape=jax.ShapeDtypeStruct((N, C), jnp.float32),
    )(*aggs, *xts)


def kernel(X, hyperedge_index, W, b):
    v = hyperedge_index[0].astype(jnp.int32)
    e = hyperedge_index[1].astype(jnp.int32)
    pad = NNZ_PAD - NNZ

    def _padded(idx, fill):
        p = jnp.concatenate([idx, jnp.full((pad,), fill, jnp.int32)])
        return p.reshape(NTILES, NCHUNK, B)

    v_g, v_s = _padded(v, 0), _padded(v, N)
    e_g, e_s = _padded(e, 0), _padded(e, N)

    zrow = jnp.zeros((RPT, G), jnp.float32)
    zcnt = jnp.zeros((RPT, 16), jnp.float32)
    ones = jnp.ones((B, 16), jnp.float32)

    xt = _matmul(X, W, b.reshape(1, C))
    s0, s1, s2, s3, cnt = _v2e(*xt, v_g, e_s, zrow, zcnt, ones)
    ys = _scale(s0, s1, s2, s3, cnt)
    aggs = _e2v(*ys, e_g, v_s, zrow)
    return _final(aggs, xt)


# speedup vs baseline: 3.2058x; 1.0495x over previous
import functools

import jax
import jax.numpy as jnp
from jax import lax
from jax.experimental import pallas as pl
from jax.experimental.pallas import tpu as pltpu
from jax.experimental.pallas import tpu_sc as plsc

N = 10000
NNZ = 160000
C = 256
G = 64
NCORES = 2
NTILES = 16
B = 256
NCHUNK = 40
PER_TILE = NCHUNK * B
NNZ_PAD = NTILES * PER_TILE
APAD = 10240
RPT = APAD // NTILES

_MESH = plsc.VectorSubcoreMesh(
    core_axis_name="c", subcore_axis_name="s",
    num_cores=NCORES, num_subcores=NTILES)


def _mm_body(x_ref, w_ref, b_ref, o0_ref, o1_ref, o2_ref, o3_ref):
    acc = jnp.dot(x_ref[...], w_ref[...],
                  preferred_element_type=jnp.float32) + b_ref[...]
    o0_ref[...] = acc[:, 0 * G:1 * G]
    o1_ref[...] = acc[:, 1 * G:2 * G]
    o2_ref[...] = acc[:, 2 * G:3 * G]
    o3_ref[...] = acc[:, 3 * G:4 * G]


def _matmul(x, w, b2):
    return pl.pallas_call(
        _mm_body,
        grid=(10,),
        in_specs=[
            pl.BlockSpec((1000, C), lambda i: (i, 0)),
            pl.BlockSpec((C, C), lambda i: (0, 0)),
            pl.BlockSpec((1, C), lambda i: (0, 0)),
        ],
        out_specs=[pl.BlockSpec((1000, G), lambda i: (i, 0))] * 4,
        out_shape=[jax.ShapeDtypeStruct((N, G), jnp.float32)] * 4,
    )(x, w, b2)


def _stream_loop(table, gidx_v, sidx_v, rows0, rows1, acc, sem0, sem1,
                 cnt_add=None):
    pltpu.async_copy(table.at[gidx_v.at[0]], rows0, sem0)
    pltpu.async_copy(table.at[gidx_v.at[1]], rows1, sem1)

    @pl.loop(0, NCHUNK, step=2)
    def _(j):
        for k, (rows, sem) in enumerate(((rows0, sem0), (rows1, sem1))):
            jj = j + k
            pltpu.make_async_copy(table.at[gidx_v.at[jj]], rows, sem).wait()
            pltpu.sync_copy(rows, acc.at[sidx_v.at[jj]], add=True)
            if cnt_add is not None:
                ones_v, cnt_acc = cnt_add
                pltpu.sync_copy(ones_v, cnt_acc.at[sidx_v.at[jj]], add=True)

            @pl.when(jj + 2 < NCHUNK)
            def _():
                pltpu.async_copy(table.at[gidx_v.at[jj + 2]], rows, sem)


def _zero_acc(zrow, acc, r0):
    pltpu.sync_copy(zrow, acc.at[pl.ds(r0, RPT)])


def _drain_acc(acc, out_hbm, r0):
    pltpu.sync_copy(acc.at[pl.ds(r0, RPT)], out_hbm.at[pl.ds(r0, RPT)])


def _v2e_body(t0, t1, t2, t3, gidx_h, sidx_h, zrow, zcnt, ones_h,
              o0, o1, o2, o3, o_cnt,
              gidx_v, sidx_v, rows0, rows1, ones_v, acc, cnt_acc, sem0, sem1):
    cid = lax.axis_index("c")
    sid = lax.axis_index("s")
    r0 = sid * RPT
    _zero_acc(zrow, acc, r0)
    pltpu.sync_copy(gidx_h.at[sid], gidx_v)
    pltpu.sync_copy(sidx_h.at[sid], sidx_v)

    @pl.when(cid == 0)
    def _():
        pltpu.sync_copy(zcnt, cnt_acc.at[pl.ds(r0, RPT)])
        pltpu.sync_copy(ones_h, ones_v)

    plsc.subcore_barrier()

    @pl.when(cid == 0)
    def _():
        _stream_loop(t0, gidx_v, sidx_v, rows0, rows1, acc, sem0, sem1,
                     cnt_add=(ones_v, cnt_acc))

    @pl.when(cid == 1)
    def _():
        _stream_loop(t2, gidx_v, sidx_v, rows0, rows1, acc, sem0, sem1)

    plsc.subcore_barrier()

    @pl.when(cid == 0)
    def _():
        _drain_acc(acc, o0, r0)
        pltpu.sync_copy(cnt_acc.at[pl.ds(r0, RPT)], o_cnt.at[pl.ds(r0, RPT)])

    @pl.when(cid == 1)
    def _():
        _drain_acc(acc, o2, r0)

    plsc.subcore_barrier()

    _zero_acc(zrow, acc, r0)
    plsc.subcore_barrier()

    @pl.when(cid == 0)
    def _():
        _stream_loop(t1, gidx_v, sidx_v, rows0, rows1, acc, sem0, sem1)

    @pl.when(cid == 1)
    def _():
        _stream_loop(t3, gidx_v, sidx_v, rows0, rows1, acc, sem0, sem1)

    plsc.subcore_barrier()

    @pl.when(cid == 0)
    def _():
        _drain_acc(acc, o1, r0)

    @pl.when(cid == 1)
    def _():
        _drain_acc(acc, o3, r0)


_v2e = functools.partial(
    pl.kernel,
    out_type=[jax.ShapeDtypeStruct((APAD, G), jnp.float32)] * 4
    + [jax.ShapeDtypeStruct((APAD, 16), jnp.float32)],
    mesh=_MESH,
    compiler_params=pltpu.CompilerParams(use_tc_tiling_on_sc=False),
    scratch_types=[
        pltpu.VMEM((NCHUNK, B), jnp.int32),
        pltpu.VMEM((NCHUNK, B), jnp.int32),
        pltpu.VMEM((B, G), jnp.float32),
        pltpu.VMEM((B, G), jnp.float32),
        pltpu.VMEM((B, 16), jnp.float32),
        pltpu.VMEM_SHARED((APAD, G), jnp.float32),
        pltpu.VMEM_SHARED((APAD, 16), jnp.float32),
        pltpu.SemaphoreType.DMA,
        pltpu.SemaphoreType.DMA,
    ],
)(_v2e_body)


def _e2v_body(t0, t1, t2, t3, gidx_h, sidx_h, zrow,
              o0, o1, o2, o3,
              gidx_v, sidx_v, rows0, rows1, acc, sem0, sem1):
    cid = lax.axis_index("c")
    sid = lax.axis_index("s")
    r0 = sid * RPT
    _zero_acc(zrow, acc, r0)
    pltpu.sync_copy(gidx_h.at[sid], gidx_v)
    pltpu.sync_copy(sidx_h.at[sid], sidx_v)

    plsc.subcore_barrier()

    @pl.when(cid == 0)
    def _():
        _stream_loop(t0, gidx_v, sidx_v, rows0, rows1, acc, sem0, sem1)

    @pl.when(cid == 1)
    def _():
        _stream_loop(t2, gidx_v, sidx_v, rows0, rows1, acc, sem0, sem1)

    plsc.subcore_barrier()

    @pl.when(cid == 0)
    def _():
        _drain_acc(acc, o0, r0)

    @pl.when(cid == 1)
    def _():
        _drain_acc(acc, o2, r0)

    plsc.subcore_barrier()

    _zero_acc(zrow, acc, r0)
    plsc.subcore_barrier()

    @pl.when(cid == 0)
    def _():
        _stream_loop(t1, gidx_v, sidx_v, rows0, rows1, acc, sem0, sem1)

    @pl.when(cid == 1)
    def _():
        _stream_loop(t3, gidx_v, sidx_v, rows0, rows1, acc, sem0, sem1)

    plsc.subcore_barrier()

    @pl.when(cid == 0)
    def _():
        _drain_acc(acc, o1, r0)

    @pl.when(cid == 1)
    def _():
        _drain_acc(acc, o3, r0)


_e2v = functools.partial(
    pl.kernel,
    out_type=[jax.ShapeDtypeStruct((APAD, G), jnp.float32)] * 4,
    mesh=_MESH,
    compiler_params=pltpu.CompilerParams(use_tc_tiling_on_sc=False),
    scratch_types=[
        pltpu.VMEM((NCHUNK, B), jnp.int32),
        pltpu.VMEM((NCHUNK, B), jnp.int32),
        pltpu.VMEM((B, G), jnp.float32),
        pltpu.VMEM((B, G), jnp.float32),
        pltpu.VMEM_SHARED((APAD, G), jnp.float32),
        pltpu.SemaphoreType.DMA,
        pltpu.SemaphoreType.DMA,
    ],
)(_e2v_body)


def _scale_body(s0, s1, s2, s3, cnt_ref, y0, y1, y2, y3):
    inv = 1.0 / jnp.maximum(cnt_ref[:, 0:1], 1.0)
    y0[...] = s0[...] * inv
    y1[...] = s1[...] * inv
    y2[...] = s2[...] * inv
    y3[...] = s3[...] * inv


def _scale(s0, s1, s2, s3, cnt):
    return pl.pallas_call(
        _scale_body,
        grid=(10,),
        in_specs=[pl.BlockSpec((1024, G), lambda i: (i, 0))] * 4
        + [pl.BlockSpec((1024, 16), lambda i: (i, 0))],
        out_specs=[pl.BlockSpec((1024, G), lambda i: (i, 0))] * 4,
        out_shape=[jax.ShapeDtypeStruct((APAD, G), jnp.float32)] * 4,
    )(s0, s1, s2, s3, cnt)


def _final_body(a0, a1, a2, a3, x0, x1, x2, x3, o_ref):
    o_ref[:, 0 * G:1 * G] = jnp.maximum(a0[...] + x0[...], 0.0)
    o_ref[:, 1 * G:2 * G] = jnp.maximum(a1[...] + x1[...], 0.0)
    o_ref[:, 2 * G:3 * G] = jnp.maximum(a2[...] + x2[...], 0.0)
    o_ref[:, 3 * G:4 * G] = jnp.maximum(a3[...] + x3[...], 0.0)


def _final(aggs, xts):
    return pl.pallas_call(
        _final_body,
        grid=(10,),
        in_specs=[pl.BlockSpec((1000, G), lambda i: (i, 0))] * 8,
        out_specs=pl.BlockSpec((1000, C), lambda i: (i, 0)),
        out_shape=jax.ShapeDtypeStruct((N, C), jnp.float32),
    )(*aggs, *xts)


def kernel(X, hyperedge_index, W, b):
    v = hyperedge_index[0].astype(jnp.int32)
    e = hyperedge_index[1].astype(jnp.int32)
    pad = NNZ_PAD - NNZ

    def _padded(idx, fill):
        p = jnp.concatenate([idx, jnp.full((pad,), fill, jnp.int32)])
        return p.reshape(NTILES, NCHUNK, B)

    v_g, v_s = _padded(v, 0), _padded(v, N)
    e_g, e_s = _padded(e, 0), _padded(e, N)

    zrow = jnp.zeros((RPT, G), jnp.float32)
    zcnt = jnp.zeros((RPT, 16), jnp.float32)
    ones = jnp.ones((B, 16), jnp.float32)

    xt = _matmul(X, W, b.reshape(1, C))
    s0, s1, s2, s3, cnt = _v2e(*xt, v_g, e_s, zrow, zcnt, ones)
    ys = _scale(s0, s1, s2, s3, cnt)
    aggs = _e2v(*ys, e_g, v_s, zrow)
    return _final(aggs, xt)


# speedup vs baseline: 3.2493x; 1.0136x over previous
import functools

import jax
import jax.numpy as jnp
from jax import lax
from jax.experimental import pallas as pl
from jax.experimental.pallas import tpu as pltpu
from jax.experimental.pallas import tpu_sc as plsc

N = 10000
NNZ = 160000
C = 256
G = 64
NCORES = 2
NTILES = 16
B = 128
NCHUNK = 80
PER_TILE = NCHUNK * B
NNZ_PAD = NTILES * PER_TILE
APAD = 10240
RPT = APAD // NTILES

_MESH = plsc.VectorSubcoreMesh(
    core_axis_name="c", subcore_axis_name="s",
    num_cores=NCORES, num_subcores=NTILES)


def _mm_body(x_ref, w_ref, b_ref, o0_ref, o1_ref, o2_ref, o3_ref):
    acc = jnp.dot(x_ref[...], w_ref[...],
                  preferred_element_type=jnp.float32) + b_ref[...]
    o0_ref[...] = acc[:, 0 * G:1 * G]
    o1_ref[...] = acc[:, 1 * G:2 * G]
    o2_ref[...] = acc[:, 2 * G:3 * G]
    o3_ref[...] = acc[:, 3 * G:4 * G]


def _matmul(x, w, b2):
    return pl.pallas_call(
        _mm_body,
        grid=(10,),
        in_specs=[
            pl.BlockSpec((1000, C), lambda i: (i, 0)),
            pl.BlockSpec((C, C), lambda i: (0, 0)),
            pl.BlockSpec((1, C), lambda i: (0, 0)),
        ],
        out_specs=[pl.BlockSpec((1000, G), lambda i: (i, 0))] * 4,
        out_shape=[jax.ShapeDtypeStruct((N, G), jnp.float32)] * 4,
    )(x, w, b2)


NBUF = 4


def _stream_loop(table, gidx_v, sidx_v, bufs, acc, cnt_add=None):
    for b, (rows, gsem, ssem) in enumerate(bufs):
        pltpu.async_copy(table.at[gidx_v.at[b]], rows, gsem)

    @pl.loop(0, NCHUNK, step=NBUF)
    def _(j):
        for b, (rows, gsem, ssem) in enumerate(bufs):
            jj = j + b
            pltpu.make_async_copy(table.at[gidx_v.at[jj]], rows, gsem).wait()
            pltpu.async_copy(rows, acc.at[sidx_v.at[jj]], ssem, add=True)
            if cnt_add is not None:
                ones_v, cnt_acc = cnt_add
                pltpu.async_copy(ones_v, cnt_acc.at[sidx_v.at[jj]], ssem,
                                 add=True)

            @pl.when(jj + NBUF < NCHUNK)
            def _():
                pltpu.make_async_copy(rows, acc.at[sidx_v.at[jj]], ssem).wait()
                if cnt_add is not None:
                    ones_v, cnt_acc = cnt_add
                    pltpu.make_async_copy(
                        ones_v, cnt_acc.at[sidx_v.at[jj]], ssem).wait()
                pltpu.async_copy(table.at[gidx_v.at[jj + NBUF]], rows, gsem)

    for b, (rows, gsem, ssem) in enumerate(bufs):
        jj = NCHUNK - NBUF + b
        pltpu.make_async_copy(rows, acc.at[sidx_v.at[jj]], ssem).wait()
        if cnt_add is not None:
            ones_v, cnt_acc = cnt_add
            pltpu.make_async_copy(ones_v, cnt_acc.at[sidx_v.at[jj]],
                                  ssem).wait()


def _zero_acc(zrow, acc, r0):
    pltpu.sync_copy(zrow, acc.at[pl.ds(r0, RPT)])


def _drain_acc(acc, out_hbm, r0):
    pltpu.sync_copy(acc.at[pl.ds(r0, RPT)], out_hbm.at[pl.ds(r0, RPT)])


def _v2e_body(t0, t1, t2, t3, gidx_h, sidx_h, zrow, zcnt, ones_h,
              o0, o1, o2, o3, o_cnt,
              gidx_v, sidx_v, r0b, r1b, r2b, r3b, ones_v, acc, cnt_acc,
              g0, g1, g2, g3, s0m, s1m, s2m, s3m):
    bufs = [(r0b, g0, s0m), (r1b, g1, s1m), (r2b, g2, s2m), (r3b, g3, s3m)]
    cid = lax.axis_index("c")
    sid = lax.axis_index("s")
    r0 = sid * RPT
    _zero_acc(zrow, acc, r0)
    pltpu.sync_copy(gidx_h.at[sid], gidx_v)
    pltpu.sync_copy(sidx_h.at[sid], sidx_v)

    @pl.when(cid == 0)
    def _():
        pltpu.sync_copy(zcnt, cnt_acc.at[pl.ds(r0, RPT)])
        pltpu.sync_copy(ones_h, ones_v)

    plsc.subcore_barrier()

    @pl.when(cid == 0)
    def _():
        _stream_loop(t0, gidx_v, sidx_v, bufs, acc,
                     cnt_add=(ones_v, cnt_acc))

    @pl.when(cid == 1)
    def _():
        _stream_loop(t2, gidx_v, sidx_v, bufs, acc)

    plsc.subcore_barrier()

    @pl.when(cid == 0)
    def _():
        _drain_acc(acc, o0, r0)
        pltpu.sync_copy(cnt_acc.at[pl.ds(r0, RPT)], o_cnt.at[pl.ds(r0, RPT)])

    @pl.when(cid == 1)
    def _():
        _drain_acc(acc, o2, r0)

    plsc.subcore_barrier()

    _zero_acc(zrow, acc, r0)
    plsc.subcore_barrier()

    @pl.when(cid == 0)
    def _():
        _stream_loop(t1, gidx_v, sidx_v, bufs, acc)

    @pl.when(cid == 1)
    def _():
        _stream_loop(t3, gidx_v, sidx_v, bufs, acc)

    plsc.subcore_barrier()

    @pl.when(cid == 0)
    def _():
        _drain_acc(acc, o1, r0)

    @pl.when(cid == 1)
    def _():
        _drain_acc(acc, o3, r0)


_v2e = functools.partial(
    pl.kernel,
    out_type=[jax.ShapeDtypeStruct((APAD, G), jnp.float32)] * 4
    + [jax.ShapeDtypeStruct((APAD, 16), jnp.float32)],
    mesh=_MESH,
    compiler_params=pltpu.CompilerParams(use_tc_tiling_on_sc=False),
    scratch_types=[
        pltpu.VMEM((NCHUNK, B), jnp.int32),
        pltpu.VMEM((NCHUNK, B), jnp.int32),
        pltpu.VMEM((B, G), jnp.float32),
        pltpu.VMEM((B, G), jnp.float32),
        pltpu.VMEM((B, G), jnp.float32),
        pltpu.VMEM((B, G), jnp.float32),
        pltpu.VMEM((B, 16), jnp.float32),
        pltpu.VMEM_SHARED((APAD, G), jnp.float32),
        pltpu.VMEM_SHARED((APAD, 16), jnp.float32),
    ] + [pltpu.SemaphoreType.DMA] * 8,
)(_v2e_body)


def _e2v_body(t0, t1, t2, t3, gidx_h, sidx_h, zrow,
              o0, o1, o2, o3,
              gidx_v, sidx_v, r0b, r1b, r2b, r3b, acc,
              g0, g1, g2, g3, s0m, s1m, s2m, s3m):
    bufs = [(r0b, g0, s0m), (r1b, g1, s1m), (r2b, g2, s2m), (r3b, g3, s3m)]
    cid = lax.axis_index("c")
    sid = lax.axis_index("s")
    r0 = sid * RPT
    _zero_acc(zrow, acc, r0)
    pltpu.sync_copy(gidx_h.at[sid], gidx_v)
    pltpu.sync_copy(sidx_h.at[sid], sidx_v)

    plsc.subcore_barrier()

    @pl.when(cid == 0)
    def _():
        _stream_loop(t0, gidx_v, sidx_v, bufs, acc)

    @pl.when(cid == 1)
    def _():
        _stream_loop(t2, gidx_v, sidx_v, bufs, acc)

    plsc.subcore_barrier()

    @pl.when(cid == 0)
    def _():
        _drain_acc(acc, o0, r0)

    @pl.when(cid == 1)
    def _():
        _drain_acc(acc, o2, r0)

    plsc.subcore_barrier()

    _zero_acc(zrow, acc, r0)
    plsc.subcore_barrier()

    @pl.when(cid == 0)
    def _():
        _stream_loop(t1, gidx_v, sidx_v, bufs, acc)

    @pl.when(cid == 1)
    def _():
        _stream_loop(t3, gidx_v, sidx_v, bufs, acc)

    plsc.subcore_barrier()

    @pl.when(cid == 0)
    def _():
        _drain_acc(acc, o1, r0)

    @pl.when(cid == 1)
    def _():
        _drain_acc(acc, o3, r0)


_e2v = functools.partial(
    pl.kernel,
    out_type=[jax.ShapeDtypeStruct((APAD, G), jnp.float32)] * 4,
    mesh=_MESH,
    compiler_params=pltpu.CompilerParams(use_tc_tiling_on_sc=False),
    scratch_types=[
        pltpu.VMEM((NCHUNK, B), jnp.int32),
        pltpu.VMEM((NCHUNK, B), jnp.int32),
        pltpu.VMEM((B, G), jnp.float32),
        pltpu.VMEM((B, G), jnp.float32),
        pltpu.VMEM((B, G), jnp.float32),
        pltpu.VMEM((B, G), jnp.float32),
        pltpu.VMEM_SHARED((APAD, G), jnp.float32),
    ] + [pltpu.SemaphoreType.DMA] * 8,
)(_e2v_body)


def _scale_body(s0, s1, s2, s3, cnt_ref, y0, y1, y2, y3):
    inv = 1.0 / jnp.maximum(cnt_ref[:, 0:1], 1.0)
    y0[...] = s0[...] * inv
    y1[...] = s1[...] * inv
    y2[...] = s2[...] * inv
    y3[...] = s3[...] * inv


def _scale(s0, s1, s2, s3, cnt):
    return pl.pallas_call(
        _scale_body,
        grid=(10,),
        in_specs=[pl.BlockSpec((1024, G), lambda i: (i, 0))] * 4
        + [pl.BlockSpec((1024, 16), lambda i: (i, 0))],
        out_specs=[pl.BlockSpec((1024, G), lambda i: (i, 0))] * 4,
        out_shape=[jax.ShapeDtypeStruct((APAD, G), jnp.float32)] * 4,
    )(s0, s1, s2, s3, cnt)


def _final_body(a0, a1, a2, a3, x0, x1, x2, x3, o_ref):
    o_ref[:, 0 * G:1 * G] = jnp.maximum(a0[...] + x0[...], 0.0)
    o_ref[:, 1 * G:2 * G] = jnp.maximum(a1[...] + x1[...], 0.0)
    o_ref[:, 2 * G:3 * G] = jnp.maximum(a2[...] + x2[...], 0.0)
    o_ref[:, 3 * G:4 * G] = jnp.maximum(a3[...] + x3[...], 0.0)


def _final(aggs, xts):
    return pl.pallas_call(
        _final_body,
        grid=(10,),
        in_specs=[pl.BlockSpec((1000, G), lambda i: (i, 0))] * 8,
        out_specs=pl.BlockSpec((1000, C), lambda i: (i, 0)),
        out_shape=jax.ShapeDtypeStruct((N, C), jnp.float32),
    )(*aggs, *xts)


def kernel(X, hyperedge_index, W, b):
    v = hyperedge_index[0].astype(jnp.int32)
    e = hyperedge_index[1].astype(jnp.int32)
    pad = NNZ_PAD - NNZ

    def _padded(idx, fill):
        p = jnp.concatenate([idx, jnp.full((pad,), fill, jnp.int32)])
        return p.reshape(NTILES, NCHUNK, B)

    v_g, v_s = _padded(v, 0), _padded(v, N)
    e_g, e_s = _padded(e, 0), _padded(e, N)

    zrow = jnp.zeros((RPT, G), jnp.float32)
    zcnt = jnp.zeros((RPT, 16), jnp.float32)
    ones = jnp.ones((B, 16), jnp.float32)

    xt = _matmul(X, W, b.reshape(1, C))
    s0, s1, s2, s3, cnt = _v2e(*xt, v_g, e_s, zrow, zcnt, ones)
    ys = _scale(s0, s1, s2, s3, cnt)
    aggs = _e2v(*ys, e_g, v_s, zrow)
    return _final(aggs, xt)


# speedup vs baseline: 5.4829x; 1.6874x over previous
import functools

import jax
import jax.numpy as jnp
from jax import lax
from jax.experimental import pallas as pl
from jax.experimental.pallas import tpu as pltpu
from jax.experimental.pallas import tpu_sc as plsc

N = 10000
NNZ = 160000
C = 256
G = 64
NCORES = 2
NTILES = 16
B = 128
NCHUNK = 80
PER_TILE = NCHUNK * B
NNZ_PAD = NTILES * PER_TILE
APAD = 10240
RPT = APAD // NTILES

_MESH = plsc.VectorSubcoreMesh(
    core_axis_name="c", subcore_axis_name="s",
    num_cores=NCORES, num_subcores=NTILES)


def _mm_body(x_ref, w_ref, b_ref, o0_ref, o1_ref, o2_ref, o3_ref):
    acc = jnp.dot(x_ref[...], w_ref[...],
                  preferred_element_type=jnp.float32) + b_ref[...]
    o0_ref[...] = acc[:, 0 * G:1 * G]
    o1_ref[...] = acc[:, 1 * G:2 * G]
    o2_ref[...] = acc[:, 2 * G:3 * G]
    o3_ref[...] = acc[:, 3 * G:4 * G]


def _matmul(x, w, b2):
    return pl.pallas_call(
        _mm_body,
        grid=(10,),
        in_specs=[
            pl.BlockSpec((1000, C), lambda i: (i, 0)),
            pl.BlockSpec((C, C), lambda i: (0, 0)),
            pl.BlockSpec((1, C), lambda i: (0, 0)),
        ],
        out_specs=[pl.BlockSpec((1000, G), lambda i: (i, 0))] * 4,
        out_shape=[jax.ShapeDtypeStruct((N, G), jnp.float32)] * 4,
    )(x, w, b2)


NBUF = 2


def _stream_loop(table, gidx_v, sidx_v, bufs, acc, cnt_add=None):
    for b, (rows, gsem, ssem) in enumerate(bufs):
        pltpu.async_copy(table.at[gidx_v.at[b]], rows, gsem)

    @pl.loop(0, NCHUNK, step=NBUF)
    def _(j):
        for b, (rows, gsem, ssem) in enumerate(bufs):
            jj = j + b
            pltpu.make_async_copy(table.at[gidx_v.at[jj]], rows, gsem).wait()
            pltpu.async_copy(rows, acc.at[sidx_v.at[jj]], ssem, add=True)
            if cnt_add is not None:
                ones_v, cnt_acc = cnt_add
                pltpu.async_copy(ones_v, cnt_acc.at[sidx_v.at[jj]], ssem,
                                 add=True)

            @pl.when(jj + NBUF < NCHUNK)
            def _():
                pltpu.make_async_copy(rows, acc.at[sidx_v.at[jj]], ssem).wait()
                if cnt_add is not None:
                    ones_v, cnt_acc = cnt_add
                    pltpu.make_async_copy(
                        ones_v, cnt_acc.at[sidx_v.at[jj]], ssem).wait()
                pltpu.async_copy(table.at[gidx_v.at[jj + NBUF]], rows, gsem)

    for b, (rows, gsem, ssem) in enumerate(bufs):
        jj = NCHUNK - NBUF + b
        pltpu.make_async_copy(rows, acc.at[sidx_v.at[jj]], ssem).wait()
        if cnt_add is not None:
            ones_v, cnt_acc = cnt_add
            pltpu.make_async_copy(ones_v, cnt_acc.at[sidx_v.at[jj]],
                                  ssem).wait()


def _zero_acc(zrow, acc, r0):
    pltpu.sync_copy(zrow, acc.at[pl.ds(r0, RPT)])


def _drain_acc(acc, out_hbm, r0):
    pltpu.sync_copy(acc.at[pl.ds(r0, RPT)], out_hbm.at[pl.ds(r0, RPT)])


def _stage_tbl(src_hbm, tbl, sid):
    @pl.when(sid < 10)
    def _():
        off = sid * 1000
        pltpu.sync_copy(src_hbm.at[pl.ds(off, 1000)], tbl.at[pl.ds(off, 1000)])


def _v2e_body(t0, t1, t2, t3, gidx_h, sidx_h, zrow, zcnt, ones_h,
              o0, o1, o2, o3, o_cnt,
              gidx_v, sidx_v, r0b, r1b, ones_v, tbl, acc, cnt_acc,
              g0, g1, s0m, s1m):
    bufs = [(r0b, g0, s0m), (r1b, g1, s1m)]
    cid = lax.axis_index("c")
    sid = lax.axis_index("s")
    r0 = sid * RPT
    _zero_acc(zrow, acc, r0)
    pltpu.sync_copy(gidx_h.at[sid], gidx_v)
    pltpu.sync_copy(sidx_h.at[sid], sidx_v)

    @pl.when(cid == 0)
    def _():
        pltpu.sync_copy(zcnt, cnt_acc.at[pl.ds(r0, RPT)])
        pltpu.sync_copy(ones_h, ones_v)
        _stage_tbl(t0, tbl, sid)

    @pl.when(cid == 1)
    def _():
        _stage_tbl(t2, tbl, sid)

    plsc.subcore_barrier()

    @pl.when(cid == 0)
    def _():
        _stream_loop(tbl, gidx_v, sidx_v, bufs, acc,
                     cnt_add=(ones_v, cnt_acc))

    @pl.when(cid == 1)
    def _():
        _stream_loop(tbl, gidx_v, sidx_v, bufs, acc)

    plsc.subcore_barrier()

    @pl.when(cid == 0)
    def _():
        _drain_acc(acc, o0, r0)
        pltpu.sync_copy(cnt_acc.at[pl.ds(r0, RPT)], o_cnt.at[pl.ds(r0, RPT)])

    @pl.when(cid == 1)
    def _():
        _drain_acc(acc, o2, r0)

    _zero_acc(zrow, acc, r0)

    @pl.when(cid == 0)
    def _():
        _stage_tbl(t1, tbl, sid)

    @pl.when(cid == 1)
    def _():
        _stage_tbl(t3, tbl, sid)

    plsc.subcore_barrier()

    @pl.when(cid == 0)
    def _():
        _stream_loop(tbl, gidx_v, sidx_v, bufs, acc)

    @pl.when(cid == 1)
    def _():
        _stream_loop(tbl, gidx_v, sidx_v, bufs, acc)

    plsc.subcore_barrier()

    @pl.when(cid == 0)
    def _():
        _drain_acc(acc, o1, r0)

    @pl.when(cid == 1)
    def _():
        _drain_acc(acc, o3, r0)


_v2e = functools.partial(
    pl.kernel,
    out_type=[jax.ShapeDtypeStruct((APAD, G), jnp.float32)] * 4
    + [jax.ShapeDtypeStruct((APAD, 16), jnp.float32)],
    mesh=_MESH,
    compiler_params=pltpu.CompilerParams(use_tc_tiling_on_sc=False),
    scratch_types=[
        pltpu.VMEM((NCHUNK, B), jnp.int32),
        pltpu.VMEM((NCHUNK, B), jnp.int32),
        pltpu.VMEM((B, G), jnp.float32),
        pltpu.VMEM((B, G), jnp.float32),
        pltpu.VMEM((B, 16), jnp.float32),
        pltpu.VMEM_SHARED((N, G), jnp.float32),
        pltpu.VMEM_SHARED((APAD, G), jnp.float32),
        pltpu.VMEM_SHARED((APAD, 16), jnp.float32),
    ] + [pltpu.SemaphoreType.DMA] * 4,
)(_v2e_body)


def _e2v_body(t0, t1, t2, t3, gidx_h, sidx_h, zrow,
              o0, o1, o2, o3,
              gidx_v, sidx_v, r0b, r1b, tbl, acc,
              g0, g1, s0m, s1m):
    bufs = [(r0b, g0, s0m), (r1b, g1, s1m)]
    cid = lax.axis_index("c")
    sid = lax.axis_index("s")
    r0 = sid * RPT
    _zero_acc(zrow, acc, r0)
    pltpu.sync_copy(gidx_h.at[sid], gidx_v)
    pltpu.sync_copy(sidx_h.at[sid], sidx_v)

    @pl.when(cid == 0)
    def _():
        _stage_tbl(t0, tbl, sid)

    @pl.when(cid == 1)
    def _():
        _stage_tbl(t2, tbl, sid)

    plsc.subcore_barrier()

    @pl.when(cid == 0)
    def _():
        _stream_loop(tbl, gidx_v, sidx_v, bufs, acc)

    @pl.when(cid == 1)
    def _():
        _stream_loop(tbl, gidx_v, sidx_v, bufs, acc)

    plsc.subcore_barrier()

    @pl.when(cid == 0)
    def _():
        _drain_acc(acc, o0, r0)

    @pl.when(cid == 1)
    def _():
        _drain_acc(acc, o2, r0)

    _zero_acc(zrow, acc, r0)

    @pl.when(cid == 0)
    def _():
        _stage_tbl(t1, tbl, sid)

    @pl.when(cid == 1)
    def _():
        _stage_tbl(t3, tbl, sid)

    plsc.subcore_barrier()

    @pl.when(cid == 0)
    def _():
        _stream_loop(tbl, gidx_v, sidx_v, bufs, acc)

    @pl.when(cid == 1)
    def _():
        _stream_loop(tbl, gidx_v, sidx_v, bufs, acc)

    plsc.subcore_barrier()

    @pl.when(cid == 0)
    def _():
        _drain_acc(acc, o1, r0)

    @pl.when(cid == 1)
    def _():
        _drain_acc(acc, o3, r0)


_e2v = functools.partial(
    pl.kernel,
    out_type=[jax.ShapeDtypeStruct((APAD, G), jnp.float32)] * 4,
    mesh=_MESH,
    compiler_params=pltpu.CompilerParams(use_tc_tiling_on_sc=False),
    scratch_types=[
        pltpu.VMEM((NCHUNK, B), jnp.int32),
        pltpu.VMEM((NCHUNK, B), jnp.int32),
        pltpu.VMEM((B, G), jnp.float32),
        pltpu.VMEM((B, G), jnp.float32),
        pltpu.VMEM_SHARED((N, G), jnp.float32),
        pltpu.VMEM_SHARED((APAD, G), jnp.float32),
    ] + [pltpu.SemaphoreType.DMA] * 4,
)(_e2v_body)


def _scale_body(s0, s1, s2, s3, cnt_ref, y0, y1, y2, y3):
    inv = 1.0 / jnp.maximum(cnt_ref[:, 0:1], 1.0)
    y0[...] = s0[...] * inv
    y1[...] = s1[...] * inv
    y2[...] = s2[...] * inv
    y3[...] = s3[...] * inv


def _scale(s0, s1, s2, s3, cnt):
    return pl.pallas_call(
        _scale_body,
        grid=(10,),
        in_specs=[pl.BlockSpec((1024, G), lambda i: (i, 0))] * 4
        + [pl.BlockSpec((1024, 16), lambda i: (i, 0))],
        out_specs=[pl.BlockSpec((1024, G), lambda i: (i, 0))] * 4,
        out_shape=[jax.ShapeDtypeStruct((APAD, G), jnp.float32)] * 4,
    )(s0, s1, s2, s3, cnt)


def _final_body(a0, a1, a2, a3, x0, x1, x2, x3, o_ref):
    o_ref[:, 0 * G:1 * G] = jnp.maximum(a0[...] + x0[...], 0.0)
    o_ref[:, 1 * G:2 * G] = jnp.maximum(a1[...] + x1[...], 0.0)
    o_ref[:, 2 * G:3 * G] = jnp.maximum(a2[...] + x2[...], 0.0)
    o_ref[:, 3 * G:4 * G] = jnp.maximum(a3[...] + x3[...], 0.0)


def _final(aggs, xts):
    return pl.pallas_call(
        _final_body,
        grid=(10,),
        in_specs=[pl.BlockSpec((1000, G), lambda i: (i, 0))] * 8,
        out_specs=pl.BlockSpec((1000, C), lambda i: (i, 0)),
        out_shape=jax.ShapeDtypeStruct((N, C), jnp.float32),
    )(*aggs, *xts)


def kernel(X, hyperedge_index, W, b):
    v = hyperedge_index[0].astype(jnp.int32)
    e = hyperedge_index[1].astype(jnp.int32)
    pad = NNZ_PAD - NNZ

    def _padded(idx, fill):
        p = jnp.concatenate([idx, jnp.full((pad,), fill, jnp.int32)])
        return p.reshape(NTILES, NCHUNK, B)

    v_g, v_s = _padded(v, 0), _padded(v, N)
    e_g, e_s = _padded(e, 0), _padded(e, N)

    zrow = jnp.zeros((RPT, G), jnp.float32)
    zcnt = jnp.zeros((RPT, 16), jnp.float32)
    ones = jnp.ones((B, 16), jnp.float32)

    xt = _matmul(X, W, b.reshape(1, C))
    s0, s1, s2, s3, cnt = _v2e(*xt, v_g, e_s, zrow, zcnt, ones)
    ys = _scale(s0, s1, s2, s3, cnt)
    aggs = _e2v(*ys, e_g, v_s, zrow)
    return _final(aggs, xt)


# speedup vs baseline: 5.9471x; 1.0847x over previous
import functools

import jax
import jax.numpy as jnp
from jax import lax
from jax.experimental import pallas as pl
from jax.experimental.pallas import tpu as pltpu
from jax.experimental.pallas import tpu_sc as plsc

N = 10000
NNZ = 160000
C = 256
G = 64
NCORES = 2
NTILES = 16
B = 128
NCHUNK = 80
PER_TILE = NCHUNK * B
NNZ_PAD = NTILES * PER_TILE
APAD = 10112
RPT = APAD // NTILES
NBUF = 2

_MESH = plsc.VectorSubcoreMesh(
    core_axis_name="c", subcore_axis_name="s",
    num_cores=NCORES, num_subcores=NTILES)


def _mm_body(x_ref, w_ref, b_ref, o0_ref, o1_ref, o2_ref, o3_ref):
    acc = jnp.dot(x_ref[...], w_ref[...],
                  preferred_element_type=jnp.float32) + b_ref[...]
    o0_ref[...] = acc[:, 0 * G:1 * G]
    o1_ref[...] = acc[:, 1 * G:2 * G]
    o2_ref[...] = acc[:, 2 * G:3 * G]
    o3_ref[...] = acc[:, 3 * G:4 * G]


def _matmul(x, w, b2):
    return pl.pallas_call(
        _mm_body,
        grid=(10,),
        in_specs=[
            pl.BlockSpec((1000, C), lambda i: (i, 0)),
            pl.BlockSpec((C, C), lambda i: (0, 0)),
            pl.BlockSpec((1, C), lambda i: (0, 0)),
        ],
        out_specs=[pl.BlockSpec((1000, G), lambda i: (i, 0))] * 4,
        out_shape=[jax.ShapeDtypeStruct((N, G), jnp.float32)] * 4,
    )(x, w, b2)


def _stream_loop(table, gidx_v, sidx_v, bufs, acc, cnt_add=None):
    for b, (rows, gsem, ssem) in enumerate(bufs):
        pltpu.async_copy(table.at[gidx_v.at[b]], rows, gsem)

    @pl.loop(0, NCHUNK, step=NBUF)
    def _(j):
        for b, (rows, gsem, ssem) in enumerate(bufs):
            jj = j + b
            pltpu.make_async_copy(table.at[gidx_v.at[jj]], rows, gsem).wait()
            pltpu.async_copy(rows, acc.at[sidx_v.at[jj]], ssem, add=True)
            if cnt_add is not None:
                ones_v, cnt_acc = cnt_add
                pltpu.async_copy(ones_v, cnt_acc.at[sidx_v.at[jj]], ssem,
                                 add=True)

            @pl.when(jj + NBUF < NCHUNK)
            def _():
                pltpu.make_async_copy(rows, acc.at[sidx_v.at[jj]], ssem).wait()
                if cnt_add is not None:
                    ones_v, cnt_acc = cnt_add
                    pltpu.make_async_copy(
                        ones_v, cnt_acc.at[sidx_v.at[jj]], ssem).wait()
                pltpu.async_copy(table.at[gidx_v.at[jj + NBUF]], rows, gsem)

    for b, (rows, gsem, ssem) in enumerate(bufs):
        jj = NCHUNK - NBUF + b
        pltpu.make_async_copy(rows, acc.at[sidx_v.at[jj]], ssem).wait()
        if cnt_add is not None:
            ones_v, cnt_acc = cnt_add
            pltpu.make_async_copy(ones_v, cnt_acc.at[sidx_v.at[jj]],
                                  ssem).wait()


def _stage_tbl(src_hbm, tbl, sid):
    @pl.when(sid < 10)
    def _():
        off = sid * 1000
        pltpu.sync_copy(src_hbm.at[pl.ds(off, 1000)], tbl.at[pl.ds(off, 1000)])


def _convert(acc, cnt_acc, tbl, rows0, ones_v, r0):
    for off, rows_n in ((0, B), (B, B), (2 * B, B), (3 * B, B), (4 * B, 120)):
        base = r0 + off
        pltpu.sync_copy(acc.at[pl.ds(base, rows_n)],
                        rows0.at[pl.ds(0, rows_n)])
        pltpu.sync_copy(cnt_acc.at[pl.ds(base, rows_n)],
                        ones_v.at[pl.ds(0, rows_n)])

        @pl.loop(0, rows_n)
        def _(r):
            inv = 1.0 / jnp.maximum(ones_v[r], 1.0)
            for k in range(G // 16):
                sl = pl.ds(k * 16, 16)
                rows0[r, sl] = rows0[r, sl] * inv

        pltpu.sync_copy(rows0.at[pl.ds(0, rows_n)],
                        tbl.at[pl.ds(base, rows_n)])


def _conv_body(t0, t1, t2, t3, vg_h, es_h, eg_h, vs_h, zrow, zcnt, ones_h,
               o0, o1, o2, o3,
               gidx_v, sidx_v, r0b, r1b, ones_v, tbl, acc, cnt_acc,
               g0, g1, s0m, s1m):
    bufs = [(r0b, g0, s0m), (r1b, g1, s1m)]
    cid = lax.axis_index("c")
    sid = lax.axis_index("s")
    r0 = sid * RPT

    for phase in (0, 1):
        pltpu.sync_copy(zrow, acc.at[pl.ds(r0, RPT)])
        pltpu.sync_copy(vg_h.at[sid], gidx_v)
        pltpu.sync_copy(es_h.at[sid], sidx_v)
        if phase == 0:
            pltpu.sync_copy(zcnt, cnt_acc.at[pl.ds(r0, RPT)])
            pltpu.sync_copy(ones_h, ones_v)

        @pl.when(cid == 0)
        def _():
            _stage_tbl(t0 if phase == 0 else t1, tbl, sid)

        @pl.when(cid == 1)
        def _():
            _stage_tbl(t2 if phase == 0 else t3, tbl, sid)

        plsc.subcore_barrier()

        if phase == 0:
            _stream_loop(tbl, gidx_v, sidx_v, bufs, acc,
                         cnt_add=(ones_v, cnt_acc))
        else:
            _stream_loop(tbl, gidx_v, sidx_v, bufs, acc)
        plsc.subcore_barrier()

        _convert(acc, cnt_acc, tbl, r0b, ones_v, r0)
        pltpu.sync_copy(zrow, acc.at[pl.ds(r0, RPT)])
        pltpu.sync_copy(eg_h.at[sid], gidx_v)
        pltpu.sync_copy(vs_h.at[sid], sidx_v)
        plsc.subcore_barrier()

        _stream_loop(tbl, gidx_v, sidx_v, bufs, acc)
        plsc.subcore_barrier()

        @pl.when(cid == 0)
        def _():
            out = o0 if phase == 0 else o1
            pltpu.sync_copy(acc.at[pl.ds(r0, RPT)], out.at[pl.ds(r0, RPT)])

        @pl.when(cid == 1)
        def _():
            out = o2 if phase == 0 else o3
            pltpu.sync_copy(acc.at[pl.ds(r0, RPT)], out.at[pl.ds(r0, RPT)])

        if phase == 0:
            plsc.subcore_barrier()


_conv = functools.partial(
    pl.kernel,
    out_type=[jax.ShapeDtypeStruct((APAD, G), jnp.float32)] * 4,
    mesh=_MESH,
    compiler_params=pltpu.CompilerParams(use_tc_tiling_on_sc=False),
    scratch_types=[
        pltpu.VMEM((NCHUNK, B), jnp.int32),
        pltpu.VMEM((NCHUNK, B), jnp.int32),
        pltpu.VMEM((B, G), jnp.float32),
        pltpu.VMEM((B, G), jnp.float32),
        pltpu.VMEM((B, 16), jnp.float32),
        pltpu.VMEM_SHARED((APAD, G), jnp.float32),
        pltpu.VMEM_SHARED((APAD, G), jnp.float32),
        pltpu.VMEM_SHARED((APAD, 16), jnp.float32),
    ] + [pltpu.SemaphoreType.DMA] * 4,
)(_conv_body)


def _final_body(a0, a1, a2, a3, x0, x1, x2, x3, o_ref):
    o_ref[:, 0 * G:1 * G] = jnp.maximum(a0[...] + x0[...], 0.0)
    o_ref[:, 1 * G:2 * G] = jnp.maximum(a1[...] + x1[...], 0.0)
    o_ref[:, 2 * G:3 * G] = jnp.maximum(a2[...] + x2[...], 0.0)
    o_ref[:, 3 * G:4 * G] = jnp.maximum(a3[...] + x3[...], 0.0)


def _final(aggs, xts):
    return pl.pallas_call(
        _final_body,
        grid=(10,),
        in_specs=[pl.BlockSpec((1000, G), lambda i: (i, 0))] * 8,
        out_specs=pl.BlockSpec((1000, C), lambda i: (i, 0)),
        out_shape=jax.ShapeDtypeStruct((N, C), jnp.float32),
    )(*aggs, *xts)


def kernel(X, hyperedge_index, W, b):
    v = hyperedge_index[0].astype(jnp.int32)
    e = hyperedge_index[1].astype(jnp.int32)
    pad = NNZ_PAD - NNZ

    def _padded(idx, fill):
        p = jnp.concatenate([idx, jnp.full((pad,), fill, jnp.int32)])
        return p.reshape(NTILES, NCHUNK, B)

    v_g, v_s = _padded(v, 0), _padded(v, N)
    e_g, e_s = _padded(e, 0), _padded(e, N)

    zrow = jnp.zeros((RPT, G), jnp.float32)
    zcnt = jnp.zeros((RPT, 16), jnp.float32)
    ones = jnp.ones((B, 16), jnp.float32)

    xt = _matmul(X, W, b.reshape(1, C))
    aggs = _conv(*xt, v_g, e_s, e_g, v_s, zrow, zcnt, ones)
    return _final(aggs, xt)
